# Initial kernel scaffold; baseline (speedup 1.0000x reference)
import jax
import jax.numpy as jnp
from jax.experimental import pallas as pl


def kernel(xyz, points, params):
    raise NotImplementedError("write your pallas kernel here")


# speedup vs baseline: 16.9422x; 16.9422x over previous
import functools

import jax
import jax.numpy as jnp
from jax import lax
from jax.experimental import pallas as pl
from jax.experimental.pallas import tpu as pltpu
from jax.experimental.pallas import tpu_sc as plsc

B = 8
N = 2048
S = 512
K = 16
DF = 32
C0 = 35
CP = 128
C1 = 64
C2 = 64
ROWS = B * S * K
CHUNK_ROWS = 8192
NCHUNK = ROWS // CHUNK_ROWS
EPS = 1e-5

SC_NC = 2
SC_NS = 16
SC_NW = SC_NC * SC_NS
RPW = ROWS // SC_NW
IDX_CHUNK = 128
NIDX = RPW // IDX_CHUNK
PHASE_ROWS = 512
NPHASE = RPW // PHASE_ROWS
DMA_PER_PHASE = PHASE_ROWS // IDX_CHUNK


def _fps_body(xyz_ref, nxyz_ref, dist_ref):
    x0 = xyz_ref[0]
    x1 = xyz_ref[1]
    x2 = xyz_ref[2]
    dist_ref[...] = jnp.full((B, N), 1e10, jnp.float32)
    lanes = lax.broadcasted_iota(jnp.int32, (B, N), 1)
    neg = jnp.float32(-3e38)

    def body(t, f):
        m = lanes == f
        cx = jnp.max(jnp.where(m, x0, neg), axis=1, keepdims=True)
        cy = jnp.max(jnp.where(m, x1, neg), axis=1, keepdims=True)
        cz = jnp.max(jnp.where(m, x2, neg), axis=1, keepdims=True)
        nxyz_ref[pl.ds(t, 1)] = jnp.concatenate([cx, cy, cz], axis=1)[None]
        d = (x0 - cx) ** 2 + (x1 - cy) ** 2 + (x2 - cz) ** 2
        dn = jnp.minimum(dist_ref[...], d)
        dist_ref[...] = dn
        mx = jnp.max(dn, axis=1, keepdims=True)
        return jnp.min(jnp.where(dn == mx, lanes, N), axis=1, keepdims=True)

    lax.fori_loop(0, S, body, jnp.zeros((B, 1), jnp.int32))


def _run_fps(xyz_planes):
    return pl.pallas_call(
        _fps_body,
        out_shape=jax.ShapeDtypeStruct((S, B, 3), jnp.float32),
        scratch_shapes=[pltpu.VMEM((B, N), jnp.float32)],
    )(xyz_planes)


def _knn_body(nxyz_ref, xyz_ref, out_ref, d_ref):
    b = pl.program_id(0)
    c = nxyz_ref[0]
    xt = xyz_ref[0]
    mm = lax.dot_general(c, xt, (((1,), (0,)), ((), ())),
                         preferred_element_type=jnp.float32)
    ss = jnp.sum(c * c, axis=1, keepdims=True)
    sq = jnp.sum(xt * xt, axis=0, keepdims=True)
    d_ref[...] = (-2.0 * mm + ss) + sq
    lanes = lax.broadcasted_iota(jnp.int32, (S, N), 1)
    cols = []
    for _ in range(K):
        v = d_ref[...]
        m = jnp.min(v, axis=1, keepdims=True)
        idx = jnp.min(jnp.where(v == m, lanes, N), axis=1, keepdims=True)
        cols.append(idx)
        d_ref[...] = jnp.where(lanes == idx, jnp.float32(3e38), v)
    out_ref[0] = jnp.concatenate(cols, axis=1) + b * N


def _run_knn(new_xyz, xyz_t):
    return pl.pallas_call(
        _knn_body,
        grid=(B,),
        in_specs=[
            pl.BlockSpec((1, S, 3), lambda b: (b, 0, 0)),
            pl.BlockSpec((1, 3, N), lambda b: (b, 0, 0)),
        ],
        out_specs=pl.BlockSpec((1, S, K), lambda b: (b, 0, 0)),
        out_shape=jax.ShapeDtypeStruct((B, S, K), jnp.int32),
        scratch_shapes=[pltpu.VMEM((S, N), jnp.float32)],
    )(new_xyz, xyz_t)


def _sc_gather_body(table_hbm, idx_hbm, out_hbm, idx_v, rows_v, sem):
    wid = lax.axis_index("s") * SC_NC + lax.axis_index("c")
    pltpu.sync_copy(idx_hbm.at[pl.ds(wid * NIDX, NIDX)], idx_v)
    for p in range(NPHASE):
        copies = []
        for j in range(DMA_PER_PHASE):
            copies.append(pltpu.async_copy(
                table_hbm.at[idx_v.at[p * DMA_PER_PHASE + j]],
                rows_v.at[pl.ds(j * IDX_CHUNK, IDX_CHUNK)],
                sem))
        for cp in copies:
            cp.wait()
        pltpu.sync_copy(rows_v,
                        out_hbm.at[pl.ds(wid * RPW + p * PHASE_ROWS, PHASE_ROWS)])


def _gather_rows(table, idx2d):
    mesh = plsc.VectorSubcoreMesh(core_axis_name="c", subcore_axis_name="s")
    kfn = pl.kernel(
        _sc_gather_body,
        mesh=mesh,
        out_type=jax.ShapeDtypeStruct((ROWS, CP), jnp.float32),
        scratch_types=[
            pltpu.VMEM((NIDX, IDX_CHUNK), jnp.int32),
            pltpu.VMEM((PHASE_ROWS, CP), jnp.float32),
            pltpu.SemaphoreType.DMA,
        ],
    )
    return kfn(table, idx2d)


def _mlp_a_body(g_ref, nx_ref, w_ref, b_ref, h_ref, s1_ref, s2_ref):
    i = pl.program_id(0)
    g = g_ref[...]
    nx = nx_ref[...]
    xn = (g.reshape(CHUNK_ROWS // K, K, CP) - nx[:, None, :]).reshape(CHUNK_ROWS, CP)
    h = lax.dot_general(xn, w_ref[...], (((1,), (0,)), ((), ())),
                        preferred_element_type=jnp.float32) + b_ref[...]
    h_ref[...] = h

    @pl.when(i == 0)
    def _():
        s1_ref[...] = jnp.zeros((1, C1), jnp.float32)
        s2_ref[...] = jnp.zeros((1, C1), jnp.float32)

    s1_ref[...] += jnp.sum(h, axis=0, keepdims=True)
    s2_ref[...] += jnp.sum(h * h, axis=0, keepdims=True)


def _run_mlp_a(gathered, nx_pad, w1p, b1):
    return pl.pallas_call(
        _mlp_a_body,
        grid=(NCHUNK,),
        in_specs=[
            pl.BlockSpec((CHUNK_ROWS, CP), lambda i: (i, 0)),
            pl.BlockSpec((CHUNK_ROWS // K, CP), lambda i: (i, 0)),
            pl.BlockSpec((CP, C1), lambda i: (0, 0)),
            pl.BlockSpec((1, C1), lambda i: (0, 0)),
        ],
        out_specs=[
            pl.BlockSpec((CHUNK_ROWS, C1), lambda i: (i, 0)),
            pl.BlockSpec((1, C1), lambda i: (0, 0)),
            pl.BlockSpec((1, C1), lambda i: (0, 0)),
        ],
        out_shape=[
            jax.ShapeDtypeStruct((ROWS, C1), jnp.float32),
            jax.ShapeDtypeStruct((1, C1), jnp.float32),
            jax.ShapeDtypeStruct((1, C1), jnp.float32),
        ],
    )(gathered, nx_pad, w1p, b1)


def _bn_relu_rows(h, s1, s2, n, g, beta):
    mean = s1 * (1.0 / n)
    var = s2 * (1.0 / n) - mean * mean
    return jnp.maximum((h - mean) / jnp.sqrt(var + EPS) * g + beta, 0.0)


def _mlp_b_body(h_ref, s1_ref, s2_ref, g_ref, be_ref, w_ref, b_ref,
                h2_ref, t1_ref, t2_ref):
    i = pl.program_id(0)
    a = _bn_relu_rows(h_ref[...], s1_ref[...], s2_ref[...], float(ROWS),
                      g_ref[...], be_ref[...])
    h2 = lax.dot_general(a, w_ref[...], (((1,), (0,)), ((), ())),
                         preferred_element_type=jnp.float32) + b_ref[...]
    h2_ref[...] = h2

    @pl.when(i == 0)
    def _():
        t1_ref[...] = jnp.zeros((1, C2), jnp.float32)
        t2_ref[...] = jnp.zeros((1, C2), jnp.float32)

    t1_ref[...] += jnp.sum(h2, axis=0, keepdims=True)
    t2_ref[...] += jnp.sum(h2 * h2, axis=0, keepdims=True)


def _run_mlp_b(h1, s1, s2, g1, beta1, w2t, b2):
    vec = pl.BlockSpec((1, C1), lambda i: (0, 0))
    return pl.pallas_call(
        _mlp_b_body,
        grid=(NCHUNK,),
        in_specs=[
            pl.BlockSpec((CHUNK_ROWS, C1), lambda i: (i, 0)),
            vec, vec, vec, vec,
            pl.BlockSpec((C1, C2), lambda i: (0, 0)),
            pl.BlockSpec((1, C2), lambda i: (0, 0)),
        ],
        out_specs=[
            pl.BlockSpec((CHUNK_ROWS, C2), lambda i: (i, 0)),
            pl.BlockSpec((1, C2), lambda i: (0, 0)),
            pl.BlockSpec((1, C2), lambda i: (0, 0)),
        ],
        out_shape=[
            jax.ShapeDtypeStruct((ROWS, C2), jnp.float32),
            jax.ShapeDtypeStruct((1, C2), jnp.float32),
            jax.ShapeDtypeStruct((1, C2), jnp.float32),
        ],
    )(h1, s1, s2, g1, beta1, w2t, b2)


def _mlp_c_body(h2_ref, t1_ref, t2_ref, g_ref, be_ref, cls_ref,
                cw1_ref, cb1_ref, cg1_ref, cbe1_ref,
                cw2_ref, cb2_ref, cg2_ref, cbe2_ref,
                out_ref, cls_out_ref):
    i = pl.program_id(0)
    a = _bn_relu_rows(h2_ref[...], t1_ref[...], t2_ref[...], float(ROWS),
                      g_ref[...], be_ref[...])
    out_ref[...] = jnp.max(a.reshape(CHUNK_ROWS // K, K, C2), axis=1)

    @pl.when(i == 0)
    def _():
        xc = cls_ref[...]
        h = lax.dot_general(xc, cw1_ref[...], (((1,), (0,)), ((), ())),
                            preferred_element_type=jnp.float32) + cb1_ref[...]
        m1 = jnp.sum(h, axis=0, keepdims=True)
        q1 = jnp.sum(h * h, axis=0, keepdims=True)
        a1 = _bn_relu_rows(h, m1, q1, float(B), cg1_ref[...], cbe1_ref[...])
        h2 = lax.dot_general(a1, cw2_ref[...], (((1,), (0,)), ((), ())),
                             preferred_element_type=jnp.float32) + cb2_ref[...]
        m2 = jnp.sum(h2, axis=0, keepdims=True)
        q2 = jnp.sum(h2 * h2, axis=0, keepdims=True)
        cls_out_ref[...] = _bn_relu_rows(h2, m2, q2, float(B),
                                         cg2_ref[...], cbe2_ref[...])


def _run_mlp_c(h2, t1, t2, g2, beta2, cls_in, cw1p, cb1, cg1, cbe1,
               cw2t, cb2, cg2, cbe2):
    vec = pl.BlockSpec((1, C2), lambda i: (0, 0))
    return pl.pallas_call(
        _mlp_c_body,
        grid=(NCHUNK,),
        in_specs=[
            pl.BlockSpec((CHUNK_ROWS, C2), lambda i: (i, 0)),
            vec, vec, vec, vec,
            pl.BlockSpec((B, CP), lambda i: (0, 0)),
            pl.BlockSpec((CP, C1), lambda i: (0, 0)),
            vec, vec, vec,
            pl.BlockSpec((C1, C2), lambda i: (0, 0)),
            vec, vec, vec,
        ],
        out_specs=[
            pl.BlockSpec((CHUNK_ROWS // K, C2), lambda i: (i, 0)),
            pl.BlockSpec((B, C2), lambda i: (0, 0)),
        ],
        out_shape=[
            jax.ShapeDtypeStruct((B * S, C2), jnp.float32),
            jax.ShapeDtypeStruct((B, C2), jnp.float32),
        ],
    )(h2, t1, t2, g2, beta2, cls_in, cw1p, cb1, cg1, cbe1,
      cw2t, cb2, cg2, cbe2)


def _pad_cols(x, cols):
    return jnp.pad(x, ((0, 0), (0, cols - x.shape[-1])))


def kernel(xyz, points, params):
    xyz_r = xyz[:, 1:, :]
    pts_r = points[:, 1:, :]

    xyz_planes = jnp.transpose(xyz_r, (2, 0, 1))
    nxyz_sb = _run_fps(xyz_planes)
    new_xyz = jnp.transpose(nxyz_sb, (1, 0, 2))

    xyz_t = jnp.transpose(xyz_r, (0, 2, 1))
    knn_flat = _run_knn(new_xyz, xyz_t)

    table = _pad_cols(
        jnp.concatenate([xyz_r, pts_r], axis=-1).reshape(B * N, C0), CP)
    gathered = _gather_rows(table, knn_flat.reshape(ROWS // IDX_CHUNK, IDX_CHUNK))

    nx_pad = _pad_cols(new_xyz.reshape(B * S, 3), CP)

    p = params
    w1p = jnp.pad(p["sa_W0"].T, ((0, CP - C0), (0, 0)))
    h1, s1, s2 = _run_mlp_a(gathered, nx_pad, w1p, p["sa_b0"][None])
    h2, t1, t2 = _run_mlp_b(h1, s1, s2, p["sa_g0"][None], p["sa_beta0"][None],
                            p["sa_W1"].T, p["sa_b1"][None])

    cls_in = _pad_cols(
        jnp.concatenate([xyz[:, 0, :], points[:, 0, :]], axis=-1), CP)
    cw1p = jnp.pad(p["cls_W0"].T, ((0, CP - C0), (0, 0)))
    sa_out, cls_out = _run_mlp_c(
        h2, t1, t2, p["sa_g1"][None], p["sa_beta1"][None],
        cls_in, cw1p, p["cls_b0"][None], p["cls_g0"][None], p["cls_beta0"][None],
        p["cls_W1"].T, p["cls_b1"][None], p["cls_g1"][None], p["cls_beta1"][None])

    xyz_out = jnp.concatenate([xyz[:, :1, :], new_xyz], axis=1)
    points_out = jnp.concatenate(
        [cls_out[:, None, :], sa_out.reshape(B, S, C2)], axis=1)
    return xyz_out, points_out


# speedup vs baseline: 19.7778x; 1.1674x over previous
import functools

import jax
import jax.numpy as jnp
from jax import lax
from jax.experimental import pallas as pl
from jax.experimental.pallas import tpu as pltpu
from jax.experimental.pallas import tpu_sc as plsc

B = 8
N = 2048
S = 512
K = 16
DF = 32
C0 = 35
CP = 128
C1 = 64
C2 = 64
ROWS = B * S * K
CHUNK_ROWS = 8192
NCHUNK = ROWS // CHUNK_ROWS
EPS = 1e-5

SC_NC = 2
SC_NS = 16
SC_NW = SC_NC * SC_NS
RPW = ROWS // SC_NW
IDX_CHUNK = 128
NIDX = RPW // IDX_CHUNK
PHASE_ROWS = 512
NPHASE = RPW // PHASE_ROWS
DMA_PER_PHASE = PHASE_ROWS // IDX_CHUNK


def _fps_body(xyz_ref, nxyz_ref):
    x0 = xyz_ref[0]
    x1 = xyz_ref[1]
    x2 = xyz_ref[2]
    lanes = lax.broadcasted_iota(jnp.int32, (B, N), 1)
    neg = jnp.float32(-3e38)

    def body(t, carry):
        dist, f = carry
        m = lanes == f
        cx = jnp.max(jnp.where(m, x0, neg), axis=1, keepdims=True)
        cy = jnp.max(jnp.where(m, x1, neg), axis=1, keepdims=True)
        cz = jnp.max(jnp.where(m, x2, neg), axis=1, keepdims=True)
        nxyz_ref[pl.ds(t, 1)] = jnp.concatenate([cx, cy, cz], axis=1)[None]
        d = (x0 - cx) ** 2 + (x1 - cy) ** 2 + (x2 - cz) ** 2
        dn = jnp.minimum(dist, d)
        nf = jnp.argmax(dn, axis=1).astype(jnp.int32)[:, None]
        return dn, nf

    lax.fori_loop(0, S, body,
                  (jnp.full((B, N), 1e10, jnp.float32),
                   jnp.zeros((B, 1), jnp.int32)))


def _run_fps(xyz_planes):
    return pl.pallas_call(
        _fps_body,
        out_shape=jax.ShapeDtypeStruct((S, B, 3), jnp.float32),
    )(xyz_planes)


def _knn_body(nxyz_ref, xyz_ref, out_ref, d_ref):
    b = pl.program_id(0)
    c = nxyz_ref[0]
    xt = xyz_ref[0]
    mm = lax.dot_general(c, xt, (((1,), (0,)), ((), ())),
                         preferred_element_type=jnp.float32)
    ss = jnp.sum(c * c, axis=1, keepdims=True)
    sq = jnp.sum(xt * xt, axis=0, keepdims=True)
    d_ref[...] = (-2.0 * mm + ss) + sq
    lanes = lax.broadcasted_iota(jnp.int32, (S, N), 1)
    cols = []
    for _ in range(K):
        v = d_ref[...]
        idx = jnp.argmin(v, axis=1).astype(jnp.int32)[:, None]
        cols.append(idx)
        d_ref[...] = jnp.where(lanes == idx, jnp.float32(3e38), v)
    out_ref[0] = jnp.concatenate(cols, axis=1) + b * N


def _run_knn(new_xyz, xyz_t):
    return pl.pallas_call(
        _knn_body,
        grid=(B,),
        in_specs=[
            pl.BlockSpec((1, S, 3), lambda b: (b, 0, 0)),
            pl.BlockSpec((1, 3, N), lambda b: (b, 0, 0)),
        ],
        out_specs=pl.BlockSpec((1, S, K), lambda b: (b, 0, 0)),
        out_shape=jax.ShapeDtypeStruct((B, S, K), jnp.int32),
        scratch_shapes=[pltpu.VMEM((S, N), jnp.float32)],
    )(new_xyz, xyz_t)


def _sc_gather_body(table_hbm, idx_hbm, out_hbm, idx_v, rows_v, sem):
    wid = lax.axis_index("s") * SC_NC + lax.axis_index("c")
    pltpu.sync_copy(idx_hbm.at[pl.ds(wid * NIDX, NIDX)], idx_v)
    for p in range(NPHASE):
        copies = []
        for j in range(DMA_PER_PHASE):
            copies.append(pltpu.async_copy(
                table_hbm.at[idx_v.at[p * DMA_PER_PHASE + j]],
                rows_v.at[pl.ds(j * IDX_CHUNK, IDX_CHUNK)],
                sem))
        for cp in copies:
            cp.wait()
        pltpu.sync_copy(rows_v,
                        out_hbm.at[pl.ds(wid * RPW + p * PHASE_ROWS, PHASE_ROWS)])


def _gather_rows(table, idx2d):
    mesh = plsc.VectorSubcoreMesh(core_axis_name="c", subcore_axis_name="s")
    kfn = pl.kernel(
        _sc_gather_body,
        mesh=mesh,
        out_type=jax.ShapeDtypeStruct((ROWS, CP), jnp.float32),
        scratch_types=[
            pltpu.VMEM((NIDX, IDX_CHUNK), jnp.int32),
            pltpu.VMEM((PHASE_ROWS, CP), jnp.float32),
            pltpu.SemaphoreType.DMA,
        ],
    )
    return kfn(table, idx2d)


def _mlp_a_body(g_ref, nx_ref, w_ref, b_ref, h_ref, s1_ref, s2_ref):
    i = pl.program_id(0)
    g = g_ref[...]
    nx = nx_ref[...]
    xn = (g.reshape(CHUNK_ROWS // K, K, CP) - nx[:, None, :]).reshape(CHUNK_ROWS, CP)
    h = lax.dot_general(xn, w_ref[...], (((1,), (0,)), ((), ())),
                        preferred_element_type=jnp.float32) + b_ref[...]
    h_ref[...] = h

    @pl.when(i == 0)
    def _():
        s1_ref[...] = jnp.zeros((1, C1), jnp.float32)
        s2_ref[...] = jnp.zeros((1, C1), jnp.float32)

    s1_ref[...] += jnp.sum(h, axis=0, keepdims=True)
    s2_ref[...] += jnp.sum(h * h, axis=0, keepdims=True)


def _run_mlp_a(gathered, nx_pad, w1p, b1):
    return pl.pallas_call(
        _mlp_a_body,
        grid=(NCHUNK,),
        in_specs=[
            pl.BlockSpec((CHUNK_ROWS, CP), lambda i: (i, 0)),
            pl.BlockSpec((CHUNK_ROWS // K, CP), lambda i: (i, 0)),
            pl.BlockSpec((CP, C1), lambda i: (0, 0)),
            pl.BlockSpec((1, C1), lambda i: (0, 0)),
        ],
        out_specs=[
            pl.BlockSpec((CHUNK_ROWS, C1), lambda i: (i, 0)),
            pl.BlockSpec((1, C1), lambda i: (0, 0)),
            pl.BlockSpec((1, C1), lambda i: (0, 0)),
        ],
        out_shape=[
            jax.ShapeDtypeStruct((ROWS, C1), jnp.float32),
            jax.ShapeDtypeStruct((1, C1), jnp.float32),
            jax.ShapeDtypeStruct((1, C1), jnp.float32),
        ],
    )(gathered, nx_pad, w1p, b1)


def _bn_relu_rows(h, s1, s2, n, g, beta):
    mean = s1 * (1.0 / n)
    var = s2 * (1.0 / n) - mean * mean
    return jnp.maximum((h - mean) / jnp.sqrt(var + EPS) * g + beta, 0.0)


def _mlp_b_body(h_ref, s1_ref, s2_ref, g_ref, be_ref, w_ref, b_ref,
                h2_ref, t1_ref, t2_ref):
    i = pl.program_id(0)
    a = _bn_relu_rows(h_ref[...], s1_ref[...], s2_ref[...], float(ROWS),
                      g_ref[...], be_ref[...])
    h2 = lax.dot_general(a, w_ref[...], (((1,), (0,)), ((), ())),
                         preferred_element_type=jnp.float32) + b_ref[...]
    h2_ref[...] = h2

    @pl.when(i == 0)
    def _():
        t1_ref[...] = jnp.zeros((1, C2), jnp.float32)
        t2_ref[...] = jnp.zeros((1, C2), jnp.float32)

    t1_ref[...] += jnp.sum(h2, axis=0, keepdims=True)
    t2_ref[...] += jnp.sum(h2 * h2, axis=0, keepdims=True)


def _run_mlp_b(h1, s1, s2, g1, beta1, w2t, b2):
    vec = pl.BlockSpec((1, C1), lambda i: (0, 0))
    return pl.pallas_call(
        _mlp_b_body,
        grid=(NCHUNK,),
        in_specs=[
            pl.BlockSpec((CHUNK_ROWS, C1), lambda i: (i, 0)),
            vec, vec, vec, vec,
            pl.BlockSpec((C1, C2), lambda i: (0, 0)),
            pl.BlockSpec((1, C2), lambda i: (0, 0)),
        ],
        out_specs=[
            pl.BlockSpec((CHUNK_ROWS, C2), lambda i: (i, 0)),
            pl.BlockSpec((1, C2), lambda i: (0, 0)),
            pl.BlockSpec((1, C2), lambda i: (0, 0)),
        ],
        out_shape=[
            jax.ShapeDtypeStruct((ROWS, C2), jnp.float32),
            jax.ShapeDtypeStruct((1, C2), jnp.float32),
            jax.ShapeDtypeStruct((1, C2), jnp.float32),
        ],
    )(h1, s1, s2, g1, beta1, w2t, b2)


def _mlp_c_body(h2_ref, t1_ref, t2_ref, g_ref, be_ref, cls_ref,
                cw1_ref, cb1_ref, cg1_ref, cbe1_ref,
                cw2_ref, cb2_ref, cg2_ref, cbe2_ref,
                out_ref, cls_out_ref):
    i = pl.program_id(0)
    a = _bn_relu_rows(h2_ref[...], t1_ref[...], t2_ref[...], float(ROWS),
                      g_ref[...], be_ref[...])
    out_ref[...] = jnp.max(a.reshape(CHUNK_ROWS // K, K, C2), axis=1)

    @pl.when(i == 0)
    def _():
        xc = cls_ref[...]
        h = lax.dot_general(xc, cw1_ref[...], (((1,), (0,)), ((), ())),
                            preferred_element_type=jnp.float32) + cb1_ref[...]
        m1 = jnp.sum(h, axis=0, keepdims=True)
        q1 = jnp.sum(h * h, axis=0, keepdims=True)
        a1 = _bn_relu_rows(h, m1, q1, float(B), cg1_ref[...], cbe1_ref[...])
        h2 = lax.dot_general(a1, cw2_ref[...], (((1,), (0,)), ((), ())),
                             preferred_element_type=jnp.float32) + cb2_ref[...]
        m2 = jnp.sum(h2, axis=0, keepdims=True)
        q2 = jnp.sum(h2 * h2, axis=0, keepdims=True)
        cls_out_ref[...] = _bn_relu_rows(h2, m2, q2, float(B),
                                         cg2_ref[...], cbe2_ref[...])


def _run_mlp_c(h2, t1, t2, g2, beta2, cls_in, cw1p, cb1, cg1, cbe1,
               cw2t, cb2, cg2, cbe2):
    vec = pl.BlockSpec((1, C2), lambda i: (0, 0))
    return pl.pallas_call(
        _mlp_c_body,
        grid=(NCHUNK,),
        in_specs=[
            pl.BlockSpec((CHUNK_ROWS, C2), lambda i: (i, 0)),
            vec, vec, vec, vec,
            pl.BlockSpec((B, CP), lambda i: (0, 0)),
            pl.BlockSpec((CP, C1), lambda i: (0, 0)),
            vec, vec, vec,
            pl.BlockSpec((C1, C2), lambda i: (0, 0)),
            vec, vec, vec,
        ],
        out_specs=[
            pl.BlockSpec((CHUNK_ROWS // K, C2), lambda i: (i, 0)),
            pl.BlockSpec((B, C2), lambda i: (0, 0)),
        ],
        out_shape=[
            jax.ShapeDtypeStruct((B * S, C2), jnp.float32),
            jax.ShapeDtypeStruct((B, C2), jnp.float32),
        ],
    )(h2, t1, t2, g2, beta2, cls_in, cw1p, cb1, cg1, cbe1,
      cw2t, cb2, cg2, cbe2)


def _pad_cols(x, cols):
    return jnp.pad(x, ((0, 0), (0, cols - x.shape[-1])))


def kernel(xyz, points, params):
    xyz_r = xyz[:, 1:, :]
    pts_r = points[:, 1:, :]

    xyz_planes = jnp.transpose(xyz_r, (2, 0, 1))
    nxyz_sb = _run_fps(xyz_planes)
    new_xyz = jnp.transpose(nxyz_sb, (1, 0, 2))

    xyz_t = jnp.transpose(xyz_r, (0, 2, 1))
    knn_flat = _run_knn(new_xyz, xyz_t)

    table = _pad_cols(
        jnp.concatenate([xyz_r, pts_r], axis=-1).reshape(B * N, C0), CP)
    gathered = _gather_rows(table, knn_flat.reshape(ROWS // IDX_CHUNK, IDX_CHUNK))

    nx_pad = _pad_cols(new_xyz.reshape(B * S, 3), CP)

    p = params
    w1p = jnp.pad(p["sa_W0"].T, ((0, CP - C0), (0, 0)))
    h1, s1, s2 = _run_mlp_a(gathered, nx_pad, w1p, p["sa_b0"][None])
    h2, t1, t2 = _run_mlp_b(h1, s1, s2, p["sa_g0"][None], p["sa_beta0"][None],
                            p["sa_W1"].T, p["sa_b1"][None])

    cls_in = _pad_cols(
        jnp.concatenate([xyz[:, 0, :], points[:, 0, :]], axis=-1), CP)
    cw1p = jnp.pad(p["cls_W0"].T, ((0, CP - C0), (0, 0)))
    sa_out, cls_out = _run_mlp_c(
        h2, t1, t2, p["sa_g1"][None], p["sa_beta1"][None],
        cls_in, cw1p, p["cls_b0"][None], p["cls_g0"][None], p["cls_beta0"][None],
        p["cls_W1"].T, p["cls_b1"][None], p["cls_g1"][None], p["cls_beta1"][None])

    xyz_out = jnp.concatenate([xyz[:, :1, :], new_xyz], axis=1)
    points_out = jnp.concatenate(
        [cls_out[:, None, :], sa_out.reshape(B, S, C2)], axis=1)
    return xyz_out, points_out
